# baseline (speedup 1.0000x reference)
import functools

import jax
import jax.numpy as jnp
from jax import lax
from jax.experimental import pallas as pl
from jax.experimental.pallas import tpu as pltpu
from jax.experimental.pallas import tpu_sc as plsc

NUM_CORES = 2
NUM_SUBCORES = 16
NUM_WORKERS = NUM_CORES * NUM_SUBCORES


def _pick_chunk(flat_per_worker: int) -> int:
    for c in (800, 400, 200, 80, 40, 16, 8):
        if flat_per_worker % c == 0:
            return c
    return flat_per_worker


def _make_sc_pooled_sum(batch, seq, vocab, dim):
    flat_per_worker = (batch * seq) // NUM_WORKERS
    b_per_w = batch // NUM_WORKERS
    chunk = _pick_chunk(flat_per_worker)
    n_chunks = flat_per_worker // chunk

    mesh = plsc.VectorSubcoreMesh(core_axis_name="c", subcore_axis_name="s")

    @functools.partial(
        pl.kernel,
        out_type=jax.ShapeDtypeStruct((batch, dim), jnp.float32),
        mesh=mesh,
        scratch_types=[
            pltpu.VMEM((chunk,), jnp.int32),
            pltpu.VMEM((chunk,), jnp.int32),
            pltpu.VMEM((chunk, dim), jnp.float32),
            pltpu.VMEM_SHARED((batch, dim), jnp.float32),
            pltpu.SemaphoreType.DMA,
        ],
        compiler_params=pltpu.CompilerParams(use_tc_tiling_on_sc=False),
    )
    def sc_pooled_sum(table_hbm, xflat_hbm, dest_hbm, zeros_hbm, out_hbm,
                      idx_v, dest_v, rows_v, acc_sh, sem):
        wid = lax.axis_index("s") * NUM_CORES + lax.axis_index("c")
        row_base = wid * b_per_w
        flat_base = wid * flat_per_worker

        pltpu.sync_copy(zeros_hbm, acc_sh.at[pl.ds(row_base, b_per_w)])

        @pl.loop(0, n_chunks)
        def _(i):
            off = flat_base + i * chunk
            pltpu.sync_copy(xflat_hbm.at[pl.ds(off, chunk)], idx_v)
            pltpu.sync_copy(dest_hbm.at[pl.ds(off, chunk)], dest_v)
            pltpu.async_copy(table_hbm.at[idx_v], rows_v, sem).wait()
            pltpu.sync_copy(rows_v, acc_sh.at[dest_v], add=True)

        pltpu.sync_copy(acc_sh.at[pl.ds(row_base, b_per_w)],
                        out_hbm.at[pl.ds(row_base, b_per_w)])

    return sc_pooled_sum


def _tc_linear(pooled_sum, w_scaled, bias):
    batch, dim = pooled_sum.shape
    out_dim = w_scaled.shape[1]

    def body(p_ref, w_ref, b_ref, o_ref):
        o_ref[...] = (
            jnp.dot(p_ref[...], w_ref[...], preferred_element_type=jnp.float32)
            + b_ref[...]
        )

    return pl.pallas_call(
        body,
        out_shape=jax.ShapeDtypeStruct((batch, out_dim), jnp.float32),
    )(pooled_sum, w_scaled, bias)


@jax.jit
def kernel(x, emb_table, fc_w, fc_b):
    batch, seq = x.shape
    vocab, dim = emb_table.shape

    xflat = x.reshape(-1).astype(jnp.int32)
    dest = jnp.repeat(
        jnp.arange(batch, dtype=jnp.int32), seq, total_repeat_length=batch * seq
    )
    zeros = jnp.zeros((batch // NUM_WORKERS, dim), jnp.float32)

    sc_fn = _make_sc_pooled_sum(batch, seq, vocab, dim)
    pooled_sum = sc_fn(emb_table, xflat, dest, zeros)

    w_scaled = fc_w.T * (1.0 / seq)
    bias = fc_b.reshape(1, -1)
    return _tc_linear(pooled_sum, w_scaled, bias)


# speedup vs baseline: 1.0028x; 1.0028x over previous
import functools

import numpy as np
import jax
import jax.numpy as jnp
from jax import lax
from jax.experimental import pallas as pl
from jax.experimental.pallas import tpu as pltpu
from jax.experimental.pallas import tpu_sc as plsc

NUM_CORES = 2
NUM_SUBCORES = 16
NUM_WORKERS = NUM_CORES * NUM_SUBCORES


def _pick_chunk(flat_per_worker: int) -> int:
    for c in (800, 400, 200, 80, 40, 16, 8):
        if flat_per_worker % c == 0:
            return c
    return flat_per_worker


def _make_sc_pooled_sum(batch, seq, vocab, dim):
    flat_per_worker = (batch * seq) // NUM_WORKERS
    b_per_w = batch // NUM_WORKERS
    chunk = _pick_chunk(flat_per_worker)
    n_chunks = flat_per_worker // chunk

    mesh = plsc.VectorSubcoreMesh(core_axis_name="c", subcore_axis_name="s")

    @functools.partial(
        pl.kernel,
        out_type=jax.ShapeDtypeStruct((batch, dim), jnp.float32),
        mesh=mesh,
        scratch_types=[
            pltpu.VMEM((chunk,), jnp.int32),
            pltpu.VMEM((chunk,), jnp.int32),
            pltpu.VMEM((chunk, dim), jnp.float32),
            pltpu.VMEM_SHARED((batch, dim), jnp.float32),
            pltpu.SemaphoreType.DMA,
        ],
        compiler_params=pltpu.CompilerParams(use_tc_tiling_on_sc=False),
    )
    def sc_pooled_sum(table_hbm, xflat_hbm, dest_hbm, zeros_hbm, out_hbm,
                      idx_v, dest_v, rows_v, acc_sh, sem):
        wid = lax.axis_index("s") * NUM_CORES + lax.axis_index("c")
        row_base = wid * b_per_w
        flat_base = wid * flat_per_worker

        pltpu.sync_copy(zeros_hbm, acc_sh.at[pl.ds(row_base, b_per_w)])

        @pl.loop(0, n_chunks)
        def _(i):
            off = flat_base + i * chunk
            pltpu.sync_copy(xflat_hbm.at[pl.ds(off, chunk)], idx_v)
            pltpu.sync_copy(dest_hbm.at[pl.ds(off, chunk)], dest_v)
            pltpu.async_copy(table_hbm.at[idx_v], rows_v, sem).wait()
            pltpu.sync_copy(rows_v, acc_sh.at[dest_v], add=True)

        pltpu.sync_copy(acc_sh.at[pl.ds(row_base, b_per_w)],
                        out_hbm.at[pl.ds(row_base, b_per_w)])

    return sc_pooled_sum


def _tc_linear(pooled_sum, w_scaled, bias):
    batch, dim = pooled_sum.shape
    out_dim = w_scaled.shape[1]

    def body(p_ref, w_ref, b_ref, o_ref):
        o_ref[...] = (
            jnp.dot(p_ref[...], w_ref[...], preferred_element_type=jnp.float32)
            + b_ref[...]
        )

    return pl.pallas_call(
        body,
        out_shape=jax.ShapeDtypeStruct((batch, out_dim), jnp.float32),
    )(pooled_sum, w_scaled, bias)


@jax.jit
def kernel(x, emb_table, fc_w, fc_b):
    batch, seq = x.shape
    vocab, dim = emb_table.shape

    xflat = x.reshape(-1).astype(jnp.int32)
    dest = jnp.asarray(np.repeat(np.arange(batch, dtype=np.int32), seq))
    zeros = jnp.asarray(np.zeros((batch // NUM_WORKERS, dim), np.float32))

    sc_fn = _make_sc_pooled_sum(batch, seq, vocab, dim)
    pooled_sum = sc_fn(emb_table, xflat, dest, zeros)

    w_scaled = fc_w.T * (1.0 / seq)
    bias = fc_b.reshape(1, -1)
    return _tc_linear(pooled_sum, w_scaled, bias)
